# Initial kernel scaffold; baseline (speedup 1.0000x reference)
import jax
import jax.numpy as jnp
from jax.experimental import pallas as pl


def kernel(q, k, topk):
    raise NotImplementedError("write your pallas kernel here")


# speedup vs baseline: 10.2813x; 10.2813x over previous
import jax
import jax.numpy as jnp
from jax import lax
from jax.experimental import pallas as pl
from jax.experimental.pallas import tpu as pltpu

_DIM = 128
_BUCKET = 64
_NEG = -3.4028234663852886e38


def _body(scale_ref, q_ref, k_ref, o_ref):
    q3 = q_ref[0]
    k3 = k_ref[0]
    nb = q3.shape[0]
    f32 = jnp.float32

    r64 = lax.broadcasted_iota(jnp.int32, (nb, _BUCKET), 0)
    c64 = lax.broadcasted_iota(jnp.int32, (nb, _BUCKET), 1)

    rinv = 1.0 / (_BUCKET * r64 + c64 + 1).astype(f32)
    m_ge = (r64 >= c64).astype(f32)
    w = jnp.dot(rinv, m_ge, preferred_element_type=f32,
                precision=lax.Precision.HIGHEST)
    h = jnp.sum(rinv, axis=1, keepdims=True)

    bq = jnp.sum(q3, axis=1)
    bk = jnp.sum(k3, axis=1)
    l_strict = (r64 > c64).astype(f32)
    pq = jnp.dot(l_strict, bq, preferred_element_type=f32,
                 precision=lax.Precision.HIGHEST)
    pk = jnp.dot(l_strict, bk, preferred_element_type=f32,
                 precision=lax.Precision.HIGHEST)

    ws = jnp.sum(k3 * w[:, :, None], axis=1)
    sk = pk * h + ws
    inv_cnt = 1.0 / (_BUCKET * r64[:, :1] + 1).astype(f32)
    sq = (pq + q3[:, 0, :]) * inv_cnt

    scale = scale_ref[0]
    r_core = lax.dot_general(sq, sk, (((1,), (1,)), ((), ())),
                             preferred_element_type=f32) * scale
    r_core = jnp.where(c64 >= r64, _NEG, r_core)

    m = jnp.maximum(jnp.max(r_core, axis=1, keepdims=True), 0.0)
    e = jnp.exp(r_core - m)
    p0 = jnp.exp(-m)
    s = p0 + jnp.sum(e, axis=1, keepdims=True)
    p_core = e / s
    p0 = p0 / s
    v = jnp.maximum(jnp.max(p_core, axis=1, keepdims=True), p0)
    cand = jnp.where(p_core == v, c64 + 1, 2 * _BUCKET)
    amin = jnp.min(cand, axis=1, keepdims=True)
    amin = jnp.where(p0 == v, 0, amin)

    ccol = lax.broadcasted_iota(jnp.int32, (nb, _DIM), 1)
    o_ref[0] = jnp.where(ccol == amin, v, 0.0)


def kernel(q, k, topk):
    bh, seq, dim = q.shape
    nb = seq // _BUCKET
    q4 = q.reshape(bh, nb, _BUCKET, dim)
    k4 = k.reshape(bh, nb, _BUCKET, dim)
    scale = (jnp.asarray(topk, jnp.float32) * (dim ** -0.5)).reshape(1)

    out = pl.pallas_call(
        _body,
        grid=(bh,),
        in_specs=[
            pl.BlockSpec(memory_space=pltpu.SMEM),
            pl.BlockSpec((1, nb, _BUCKET, dim), lambda b: (b, 0, 0, 0)),
            pl.BlockSpec((1, nb, _BUCKET, dim), lambda b: (b, 0, 0, 0)),
        ],
        out_specs=pl.BlockSpec((1, nb, _DIM), lambda b: (b, 0, 0)),
        out_shape=jax.ShapeDtypeStruct((bh, nb, _DIM), jnp.float32),
        compiler_params=pltpu.CompilerParams(
            dimension_semantics=("arbitrary",),
        ),
    )(scale, q4, k4)
    return out[:, :, : nb + 1]
